# baseline (speedup 1.0000x reference)
import jax
import jax.numpy as jnp
from jax.experimental import pallas as pl
from jax.experimental.pallas import tpu as pltpu

_C, _H, _W = 3, 512, 512
_RCH = 64
_NCH = _H // _RCH


def _norm_kernel(x_ref, o_ref):
    acc_s = jnp.zeros((_RCH, _W), jnp.float32)
    acc_c = jnp.zeros((_RCH, _W), jnp.float32)
    for c in range(_C):
        for k in range(_NCH):
            blk = x_ref[0, c, k * _RCH:(k + 1) * _RCH, :]
            acc_s = acc_s + blk
            acc_c = acc_c + jnp.where(blk != 0.0, 1.0, 0.0)
    cnt = jnp.sum(acc_c)
    mean = jnp.sum(acc_s) / cnt

    acc_v = jnp.zeros((_RCH, _W), jnp.float32)
    for c in range(_C):
        for k in range(_NCH):
            blk = x_ref[0, c, k * _RCH:(k + 1) * _RCH, :]
            d = blk - mean
            acc_v = acc_v + jnp.where(blk != 0.0, d * d, 0.0)
    var = jnp.sum(acc_v) / (cnt - 1.0)
    inv = jax.lax.rsqrt(var)
    shift = -mean * inv

    for c in range(_C):
        for k in range(_NCH):
            blk = x_ref[0, c, k * _RCH:(k + 1) * _RCH, :]
            o_ref[0, c, k * _RCH:(k + 1) * _RCH, :] = jnp.where(
                blk != 0.0, blk * inv + shift, blk)


def kernel(tensor):
    b, ch, h, w = tensor.shape
    return pl.pallas_call(
        _norm_kernel,
        grid=(b,),
        in_specs=[pl.BlockSpec((1, ch, h, w), lambda i: (i, 0, 0, 0))],
        out_specs=pl.BlockSpec((1, ch, h, w), lambda i: (i, 0, 0, 0)),
        out_shape=jax.ShapeDtypeStruct((b, ch, h, w), jnp.float32),
        compiler_params=pltpu.CompilerParams(
            dimension_semantics=("parallel",),
            vmem_limit_bytes=48 * 1024 * 1024,
        ),
        name="masked_sample_norm",
    )(tensor)


# speedup vs baseline: 1.1058x; 1.1058x over previous
import jax
import jax.numpy as jnp
from jax.experimental import pallas as pl
from jax.experimental.pallas import tpu as pltpu

_C, _H, _W = 3, 512, 512
_RCH = 64
_NCH = _H // _RCH


def _norm_kernel(x_ref, o_ref):
    acc_s = jnp.zeros((8, _W), jnp.float32)
    acc_q = jnp.zeros((8, _W), jnp.float32)
    acc_c = jnp.zeros((8, _W), jnp.float32)
    for c in range(_C):
        for k in range(_NCH):
            blk = x_ref[0, c, k * _RCH:(k + 1) * _RCH, :]
            b3 = blk.reshape(_RCH // 8, 8, _W)
            acc_s = acc_s + jnp.sum(b3, axis=0)
            acc_q = acc_q + jnp.sum(b3 * b3, axis=0)
            acc_c = acc_c + jnp.sum(jnp.where(b3 != 0.0, 1.0, 0.0), axis=0)
    cnt = jnp.sum(acc_c)
    mean = jnp.sum(acc_s) / cnt
    var = (jnp.sum(acc_q) - cnt * mean * mean) / (cnt - 1.0)
    inv = jax.lax.rsqrt(var)
    shift = -mean * inv

    for c in range(_C):
        for k in range(_NCH):
            blk = x_ref[0, c, k * _RCH:(k + 1) * _RCH, :]
            o_ref[0, c, k * _RCH:(k + 1) * _RCH, :] = jnp.where(
                blk != 0.0, blk * inv + shift, blk)


def kernel(tensor):
    b, ch, h, w = tensor.shape
    return pl.pallas_call(
        _norm_kernel,
        grid=(b,),
        in_specs=[pl.BlockSpec((1, ch, h, w), lambda i: (i, 0, 0, 0))],
        out_specs=pl.BlockSpec((1, ch, h, w), lambda i: (i, 0, 0, 0)),
        out_shape=jax.ShapeDtypeStruct((b, ch, h, w), jnp.float32),
        compiler_params=pltpu.CompilerParams(
            dimension_semantics=("parallel",),
            vmem_limit_bytes=48 * 1024 * 1024,
        ),
        name="masked_sample_norm",
    )(tensor)


# speedup vs baseline: 1.1067x; 1.0008x over previous
import jax
import jax.numpy as jnp
from jax.experimental import pallas as pl
from jax.experimental.pallas import tpu as pltpu

_C, _H, _W = 3, 512, 512
_RCH = 32
_NCH = _H // _RCH


def _norm_kernel(x_ref, o_ref):
    acc_s = jnp.zeros((8, _W), jnp.float32)
    acc_q = jnp.zeros((8, _W), jnp.float32)
    acc_c = jnp.zeros((8, _W), jnp.float32)
    for c in range(_C):
        for k in range(_NCH):
            blk = x_ref[0, c, k * _RCH:(k + 1) * _RCH, :]
            b3 = blk.reshape(_RCH // 8, 8, _W)
            acc_s = acc_s + jnp.sum(b3, axis=0)
            acc_q = acc_q + jnp.sum(b3 * b3, axis=0)
            acc_c = acc_c + jnp.sum(jnp.where(b3 != 0.0, 1.0, 0.0), axis=0)
    cnt = jnp.sum(acc_c)
    mean = jnp.sum(acc_s) / cnt
    var = (jnp.sum(acc_q) - cnt * mean * mean) / (cnt - 1.0)
    inv = jax.lax.rsqrt(var)
    shift = -mean * inv

    for c in range(_C):
        for k in range(_NCH):
            blk = x_ref[0, c, k * _RCH:(k + 1) * _RCH, :]
            o_ref[0, c, k * _RCH:(k + 1) * _RCH, :] = jnp.where(
                blk != 0.0, blk * inv + shift, blk)


def kernel(tensor):
    b, ch, h, w = tensor.shape
    return pl.pallas_call(
        _norm_kernel,
        grid=(b,),
        in_specs=[pl.BlockSpec((1, ch, h, w), lambda i: (i, 0, 0, 0))],
        out_specs=pl.BlockSpec((1, ch, h, w), lambda i: (i, 0, 0, 0)),
        out_shape=jax.ShapeDtypeStruct((b, ch, h, w), jnp.float32),
        compiler_params=pltpu.CompilerParams(
            dimension_semantics=("parallel",),
            vmem_limit_bytes=48 * 1024 * 1024,
        ),
        name="masked_sample_norm",
    )(tensor)


# speedup vs baseline: 1.2467x; 1.1265x over previous
import jax
import jax.numpy as jnp
from jax.experimental import pallas as pl
from jax.experimental.pallas import tpu as pltpu

_C, _H, _W = 3, 512, 512
_RCH = 32
_NCH = _H // _RCH
_SPB = 2


def _norm_kernel(x_ref, o_ref):
    for s in range(x_ref.shape[0]):
        _one_sample(x_ref, o_ref, s)


def _one_sample(x_ref, o_ref, s):
    acc_s = jnp.zeros((8, _W), jnp.float32)
    acc_q = jnp.zeros((8, _W), jnp.float32)
    acc_c = jnp.zeros((8, _W), jnp.float32)
    for c in range(_C):
        for k in range(_NCH):
            blk = x_ref[s, c, k * _RCH:(k + 1) * _RCH, :]
            b3 = blk.reshape(_RCH // 8, 8, _W)
            acc_s = acc_s + jnp.sum(b3, axis=0)
            acc_q = acc_q + jnp.sum(b3 * b3, axis=0)
            acc_c = acc_c + jnp.sum(jnp.where(b3 != 0.0, 1.0, 0.0), axis=0)
    cnt = jnp.sum(acc_c)
    mean = jnp.sum(acc_s) / cnt
    var = (jnp.sum(acc_q) - cnt * mean * mean) / (cnt - 1.0)
    inv = jax.lax.rsqrt(var)
    shift = -mean * inv

    for c in range(_C):
        for k in range(_NCH):
            blk = x_ref[s, c, k * _RCH:(k + 1) * _RCH, :]
            o_ref[s, c, k * _RCH:(k + 1) * _RCH, :] = jnp.where(
                blk != 0.0, blk * inv + shift, blk)


def kernel(tensor):
    b, ch, h, w = tensor.shape
    return pl.pallas_call(
        _norm_kernel,
        grid=(b // _SPB,),
        in_specs=[pl.BlockSpec((_SPB, ch, h, w), lambda i: (i, 0, 0, 0))],
        out_specs=pl.BlockSpec((_SPB, ch, h, w), lambda i: (i, 0, 0, 0)),
        out_shape=jax.ShapeDtypeStruct((b, ch, h, w), jnp.float32),
        compiler_params=pltpu.CompilerParams(
            dimension_semantics=("parallel",),
            vmem_limit_bytes=48 * 1024 * 1024,
        ),
        name="masked_sample_norm",
    )(tensor)


# speedup vs baseline: 1.2522x; 1.0044x over previous
import jax
import jax.numpy as jnp
from jax.experimental import pallas as pl
from jax.experimental.pallas import tpu as pltpu

_C, _H, _W = 3, 512, 512
_RCH = 32
_NCH = _H // _RCH
_SPB = 4


def _norm_kernel(x_ref, o_ref):
    for s in range(x_ref.shape[0]):
        _one_sample(x_ref, o_ref, s)


def _one_sample(x_ref, o_ref, s):
    acc_s = jnp.zeros((8, _W), jnp.float32)
    acc_q = jnp.zeros((8, _W), jnp.float32)
    acc_c = jnp.zeros((8, _W), jnp.float32)
    for c in range(_C):
        for k in range(_NCH):
            blk = x_ref[s, c, k * _RCH:(k + 1) * _RCH, :]
            b3 = blk.reshape(_RCH // 8, 8, _W)
            acc_s = acc_s + jnp.sum(b3, axis=0)
            acc_q = acc_q + jnp.sum(b3 * b3, axis=0)
            acc_c = acc_c + jnp.sum(jnp.where(b3 != 0.0, 1.0, 0.0), axis=0)
    cnt = jnp.sum(acc_c)
    mean = jnp.sum(acc_s) / cnt
    var = (jnp.sum(acc_q) - cnt * mean * mean) / (cnt - 1.0)
    inv = jax.lax.rsqrt(var)
    shift = -mean * inv

    for c in range(_C):
        for k in range(_NCH):
            blk = x_ref[s, c, k * _RCH:(k + 1) * _RCH, :]
            o_ref[s, c, k * _RCH:(k + 1) * _RCH, :] = jnp.where(
                blk != 0.0, blk * inv + shift, blk)


def kernel(tensor):
    b, ch, h, w = tensor.shape
    return pl.pallas_call(
        _norm_kernel,
        grid=(b // _SPB,),
        in_specs=[pl.BlockSpec((_SPB, ch, h, w), lambda i: (i, 0, 0, 0))],
        out_specs=pl.BlockSpec((_SPB, ch, h, w), lambda i: (i, 0, 0, 0)),
        out_shape=jax.ShapeDtypeStruct((b, ch, h, w), jnp.float32),
        compiler_params=pltpu.CompilerParams(
            dimension_semantics=("parallel",),
            vmem_limit_bytes=56 * 1024 * 1024,
        ),
        name="masked_sample_norm",
    )(tensor)
